# baseline (speedup 1.0000x reference)
import functools

import jax
import jax.numpy as jnp
from jax import lax
from jax.experimental import pallas as pl
from jax.experimental.pallas import tpu as pltpu
from jax.experimental.pallas import tpu_sc as plsc

VOCAB = 1000000
EMB_DIM = 48
BATCH = 16384
H1 = 64
H2 = 32

NC = 2
NS = 16
NW = NC * NS
B_PER_W = BATCH // NW
CHUNK = 128
NCHUNK = B_PER_W // CHUNK


def _sc_gather(table, idx):
    mesh = plsc.VectorSubcoreMesh(core_axis_name="c", subcore_axis_name="s")
    NSEM = 4
    HALF = B_PER_W // 2

    @functools.partial(
        pl.kernel,
        mesh=mesh,
        out_type=jax.ShapeDtypeStruct((BATCH, EMB_DIM), jnp.float32),
        scratch_types=[
            pltpu.VMEM((B_PER_W,), jnp.int32),
            pltpu.VMEM((B_PER_W, EMB_DIM), jnp.float32),
        ] + [pltpu.SemaphoreType.DMA] * NSEM,
    )
    def gather_kernel(table_hbm, idx_hbm, out_hbm, idx_v, rows_v, *sems):
        wid = lax.axis_index("s") * NC + lax.axis_index("c")
        base = wid * B_PER_W
        pltpu.sync_copy(idx_hbm.at[pl.ds(base, B_PER_W)], idx_v)

        def body(k, carry):
            v = idx_v[pl.ds(k * 16, 16)]
            for j in range(16):
                r = v[j]
                pltpu.async_copy(
                    table_hbm.at[pl.ds(r, 1)],
                    rows_v.at[pl.ds(k * 16 + j, 1)],
                    sems[j % NSEM],
                )
            return carry

        lax.fori_loop(0, B_PER_W // 16, body, 0)
        for m in range(NSEM):
            pltpu.make_async_copy(
                table_hbm.at[pl.ds(0, B_PER_W // NSEM)],
                rows_v.at[pl.ds(0, B_PER_W // NSEM)],
                sems[m],
            ).wait()
        pltpu.sync_copy(rows_v, out_hbm.at[pl.ds(base, B_PER_W)])

    return gather_kernel(table, idx)


CP_BLK = 8192


def _tc_copy(table):
    def body(in_ref, out_ref):
        out_ref[...] = in_ref[...]

    return pl.pallas_call(
        body,
        grid=(VOCAB // CP_BLK,),
        in_specs=[pl.BlockSpec((CP_BLK, EMB_DIM), lambda i: (i, 0))],
        out_specs=pl.BlockSpec((CP_BLK, EMB_DIM), lambda i: (i, 0)),
        out_shape=jax.ShapeDtypeStruct((VOCAB, EMB_DIM), jnp.float32),
    )(table)


MLP_BLK = 2048


def _mlp_body(emb_ref, w1_ref, b1_ref, w2_ref, b2_ref, out_ref):
    h = jnp.dot(emb_ref[...], w1_ref[...], preferred_element_type=jnp.float32)
    h = jnp.maximum(h + b1_ref[...], 0.0)
    out_ref[...] = (
        jnp.dot(h, w2_ref[...], preferred_element_type=jnp.float32) + b2_ref[...]
    )


def _tc_mlp(emb, W1, b1, W2, b2):
    grid = (BATCH // MLP_BLK,)
    return pl.pallas_call(
        _mlp_body,
        grid=grid,
        in_specs=[
            pl.BlockSpec((MLP_BLK, EMB_DIM), lambda i: (i, 0)),
            pl.BlockSpec((EMB_DIM, H1), lambda i: (0, 0)),
            pl.BlockSpec((1, H1), lambda i: (0, 0)),
            pl.BlockSpec((H1, H2), lambda i: (0, 0)),
            pl.BlockSpec((1, H2), lambda i: (0, 0)),
        ],
        out_specs=pl.BlockSpec((MLP_BLK, H2), lambda i: (i, 0)),
        out_shape=jax.ShapeDtypeStruct((BATCH, H2), jnp.float32),
    )(emb, W1, b1.reshape(1, H1), W2, b2.reshape(1, H2))


def kernel(user_id, table, W1, b1, W2, b2):
    table2 = _tc_copy(table)
    emb = _sc_gather(table2, user_id.astype(jnp.int32))
    return _tc_mlp(emb, W1, b1, W2, b2)


# speedup vs baseline: 3.9263x; 3.9263x over previous
import functools

import jax
import jax.numpy as jnp
from jax import lax
from jax.experimental import pallas as pl
from jax.experimental.pallas import tpu as pltpu
from jax.experimental.pallas import tpu_sc as plsc

VOCAB = 1000000
EMB_DIM = 48
BATCH = 16384
H1 = 64
H2 = 32

NC = 2
NS = 16
COLS_PER_CORE = EMB_DIM // NC
B_PER_SUB = BATCH // NS
CHUNK = 128
NCHUNK = B_PER_SUB // CHUNK


def _sc_gather_cols(tableT, idx2d):
    mesh = plsc.VectorSubcoreMesh(core_axis_name="c", subcore_axis_name="s")

    @functools.partial(
        pl.kernel,
        mesh=mesh,
        out_type=jax.ShapeDtypeStruct((EMB_DIM, BATCH), jnp.float32),
        scratch_types=[
            pltpu.VMEM((NCHUNK, CHUNK), jnp.int32),
            pltpu.VMEM((1, B_PER_SUB), jnp.float32),
            pltpu.VMEM_SHARED((VOCAB,), jnp.float32),
            pltpu.SemaphoreType.DMA,
        ],
    )
    def gather_kernel(tableT_hbm, idx_hbm, outT_hbm, idx_v, vals_v, sp_row,
                      sem):
        cid = lax.axis_index("c")
        sid = lax.axis_index("s")
        pltpu.sync_copy(idx_hbm.at[pl.ds(sid * NCHUNK, NCHUNK)], idx_v)

        def col_body(i, carry):
            col = cid * COLS_PER_CORE + i

            @pl.when(sid == 0)
            def _load():
                pltpu.sync_copy(tableT_hbm.at[col], sp_row)

            plsc.subcore_barrier()
            copies = []
            for j in range(NCHUNK):
                copies.append(
                    pltpu.async_copy(
                        sp_row.at[idx_v.at[j]],
                        vals_v.at[0, pl.ds(j * CHUNK, CHUNK)],
                        sem,
                    )
                )
            for cp in copies:
                cp.wait()
            pltpu.sync_copy(
                vals_v,
                outT_hbm.at[pl.ds(col, 1), pl.ds(sid * B_PER_SUB, B_PER_SUB)],
            )
            plsc.subcore_barrier()
            return carry

        lax.fori_loop(0, COLS_PER_CORE, col_body, 0)

    return gather_kernel(tableT, idx2d)


MLP_BLK = 2048


def _mlp_t_body(embT_ref, w1_ref, b1_ref, w2_ref, b2_ref, outT_ref):
    hT = lax.dot_general(
        w1_ref[...], embT_ref[...], (((0,), (0,)), ((), ())),
        preferred_element_type=jnp.float32,
    )
    hT = jnp.maximum(hT + b1_ref[...], 0.0)
    outT_ref[...] = (
        lax.dot_general(
            w2_ref[...], hT, (((0,), (0,)), ((), ())),
            preferred_element_type=jnp.float32,
        )
        + b2_ref[...]
    )


def _tc_mlp_t(embT, W1, b1, W2, b2):
    grid = (BATCH // MLP_BLK,)
    return pl.pallas_call(
        _mlp_t_body,
        grid=grid,
        in_specs=[
            pl.BlockSpec((EMB_DIM, MLP_BLK), lambda i: (0, i)),
            pl.BlockSpec((EMB_DIM, H1), lambda i: (0, 0)),
            pl.BlockSpec((H1, 1), lambda i: (0, 0)),
            pl.BlockSpec((H1, H2), lambda i: (0, 0)),
            pl.BlockSpec((H2, 1), lambda i: (0, 0)),
        ],
        out_specs=pl.BlockSpec((H2, MLP_BLK), lambda i: (0, i)),
        out_shape=jax.ShapeDtypeStruct((H2, BATCH), jnp.float32),
    )(embT, W1, b1.reshape(H1, 1), W2, b2.reshape(H2, 1))


def kernel(user_id, table, W1, b1, W2, b2):
    idx2d = user_id.astype(jnp.int32).reshape(BATCH // CHUNK, CHUNK)
    tableT = table.T
    embT = _sc_gather_cols(tableT, idx2d)
    outT = _tc_mlp_t(embT, W1, b1, W2, b2)
    return outT.T


# speedup vs baseline: 4.1035x; 1.0451x over previous
import functools

import jax
import jax.numpy as jnp
from jax import lax
from jax.experimental import pallas as pl
from jax.experimental.pallas import tpu as pltpu
from jax.experimental.pallas import tpu_sc as plsc

VOCAB = 1000000
EMB_DIM = 48
BATCH = 16384
H1 = 64
H2 = 32

NC = 2
NS = 16
COLS_PER_CORE = EMB_DIM // NC
B_PER_SUB = BATCH // NS
CHUNK = 128
NCHUNK = B_PER_SUB // CHUNK


def _sc_gather_cols(tableT, idx2d):
    mesh = plsc.VectorSubcoreMesh(core_axis_name="c", subcore_axis_name="s")

    @functools.partial(
        pl.kernel,
        mesh=mesh,
        out_type=jax.ShapeDtypeStruct((EMB_DIM, BATCH), jnp.float32),
        scratch_types=[
            pltpu.VMEM((NCHUNK, CHUNK), jnp.int32),
            pltpu.VMEM((1, B_PER_SUB), jnp.float32),
            pltpu.VMEM_SHARED((VOCAB,), jnp.float32),
            pltpu.VMEM_SHARED((VOCAB,), jnp.float32),
            pltpu.SemaphoreType.DMA,
            pltpu.SemaphoreType.DMA,
        ],
    )
    def gather_kernel(tableT_hbm, idx_hbm, outT_hbm, idx_v, vals_v, sp_a,
                      sp_b, sem, sem_l):
        cid = lax.axis_index("c")
        sid = lax.axis_index("s")
        c0 = cid * COLS_PER_CORE
        pltpu.sync_copy(idx_hbm.at[pl.ds(sid * NCHUNK, NCHUNK)], idx_v)

        bufs = (sp_a, sp_b)

        @pl.when(sid == 0)
        def _preload():
            pltpu.sync_copy(tableT_hbm.at[c0], sp_a)

        plsc.subcore_barrier()

        for i in range(COLS_PER_CORE):
            cur = bufs[i % 2]
            nxt = bufs[(i + 1) % 2]
            if i + 1 < COLS_PER_CORE:
                @pl.when(sid == 0)
                def _start_next(i=i, nxt=nxt):
                    pltpu.async_copy(tableT_hbm.at[c0 + i + 1], nxt, sem_l)

            copies = []
            for j in range(NCHUNK):
                copies.append(
                    pltpu.async_copy(
                        cur.at[idx_v.at[j]],
                        vals_v.at[0, pl.ds(j * CHUNK, CHUNK)],
                        sem,
                    )
                )
            for cp in copies:
                cp.wait()
            pltpu.sync_copy(
                vals_v,
                outT_hbm.at[
                    pl.ds(c0 + i, 1), pl.ds(sid * B_PER_SUB, B_PER_SUB)
                ],
            )
            if i + 1 < COLS_PER_CORE:
                @pl.when(sid == 0)
                def _wait_next(i=i, nxt=nxt):
                    pltpu.make_async_copy(
                        tableT_hbm.at[c0 + i + 1], nxt, sem_l
                    ).wait()

            plsc.subcore_barrier()

    return gather_kernel(tableT, idx2d)


MLP_BLK = 2048


def _mlp_t_body(embT_ref, w1_ref, b1_ref, w2_ref, b2_ref, outT_ref):
    hT = lax.dot_general(
        w1_ref[...], embT_ref[...], (((0,), (0,)), ((), ())),
        preferred_element_type=jnp.float32,
    )
    hT = jnp.maximum(hT + b1_ref[...], 0.0)
    outT_ref[...] = (
        lax.dot_general(
            w2_ref[...], hT, (((0,), (0,)), ((), ())),
            preferred_element_type=jnp.float32,
        )
        + b2_ref[...]
    )


def _tc_mlp_t(embT, W1, b1, W2, b2):
    grid = (BATCH // MLP_BLK,)
    return pl.pallas_call(
        _mlp_t_body,
        grid=grid,
        in_specs=[
            pl.BlockSpec((EMB_DIM, MLP_BLK), lambda i: (0, i)),
            pl.BlockSpec((EMB_DIM, H1), lambda i: (0, 0)),
            pl.BlockSpec((H1, 1), lambda i: (0, 0)),
            pl.BlockSpec((H1, H2), lambda i: (0, 0)),
            pl.BlockSpec((H2, 1), lambda i: (0, 0)),
        ],
        out_specs=pl.BlockSpec((H2, MLP_BLK), lambda i: (0, i)),
        out_shape=jax.ShapeDtypeStruct((H2, BATCH), jnp.float32),
    )(embT, W1, b1.reshape(H1, 1), W2, b2.reshape(H2, 1))


def kernel(user_id, table, W1, b1, W2, b2):
    idx2d = user_id.astype(jnp.int32).reshape(BATCH // CHUNK, CHUNK)
    tableT = table.T
    embT = _sc_gather_cols(tableT, idx2d)
    outT = _tc_mlp_t(embT, W1, b1, W2, b2)
    return outT.T
